# baseline (speedup 1.0000x reference)
import functools
import math

import numpy as np
import jax
import jax.numpy as jnp
from jax import lax
from jax.experimental import pallas as pl
from jax.experimental.pallas import tpu as pltpu
from jax.experimental.pallas import tpu_sc as plsc

VOCAB = 100000
DIM = 128
MAX_LEN = 200
BATCH = 1024
SEQ = 200

N = BATCH * SEQ
NW = 32
ROWS_PER_W = N // NW
CHUNK = 128
NCHUNK = ROWS_PER_W // CHUNK
NSLOT = 5
NCOMB = 3 * SEQ
CROWS = 40


def _pe_table_np():
    position = np.arange(MAX_LEN, dtype=np.float32)[:, None]
    div_term = np.exp(
        np.arange(0, DIM, 2, dtype=np.float32) * -(math.log(10000.0) / DIM)
    )
    pe = np.zeros((MAX_LEN, DIM), dtype=np.float32)
    pe[:, 0::2] = np.sin(position * div_term)
    pe[:, 1::2] = np.cos(position * div_term)
    return pe


_PE_NP = _pe_table_np()
_POS_NP = (np.arange(ROWS_PER_W, dtype=np.int32) % SEQ)


def _sc_body(t_hbm, s_hbm, pe_hbm, pos_hbm, tok_hbm, seg_hbm, out_hbm,
             comb_sh, idx_t, idx_c, pos_v, s_v, cbuf, bufs,
             sems_c, sems_t, sems_w):
    cid = lax.axis_index("c")
    sid = lax.axis_index("s")
    wid = sid * 2 + cid
    obase = wid * ROWS_PER_W

    def islice(ref, g, b):
        off = pl.multiple_of((g * NSLOT + b) * CHUNK, CHUNK)
        return ref.at[pl.ds(off, CHUNK)]

    pltpu.sync_copy(tok_hbm.at[pl.ds(obase, ROWS_PER_W)], idx_t)
    t0 = [
        pltpu.async_copy(t_hbm.at[islice(idx_t, 0, b)], bufs[b], sems_t[b])
        for b in range(NSLOT)
    ]

    @pl.when(sid < 15)
    def _():
        r0 = pl.multiple_of(sid * CROWS, CROWS)
        seg_id = r0 // SEQ
        pos0 = pl.multiple_of(r0 - seg_id * SEQ, CROWS)
        pltpu.sync_copy(pe_hbm.at[pl.ds(pos0, CROWS)], cbuf)
        pltpu.sync_copy(s_hbm, s_v)

        def row_fn(r, c):
            for k in range(DIM // 16):
                sl = pl.ds(k * 16, 16)
                cbuf[r, sl] = cbuf[r, sl] + s_v[seg_id, sl]
            return c

        lax.fori_loop(0, CROWS, row_fn, 0)
        pltpu.sync_copy(cbuf, comb_sh.at[pl.ds(r0, CROWS)])

    pltpu.sync_copy(seg_hbm.at[pl.ds(obase, ROWS_PER_W)], idx_c)
    pltpu.sync_copy(pos_hbm, pos_v)

    def cidx_fn(i, c):
        sl = pl.ds(pl.multiple_of(i * 16, 16), 16)
        idx_c[sl] = idx_c[sl] * SEQ + pos_v[sl]
        return c

    lax.fori_loop(0, ROWS_PER_W // 16, cidx_fn, 0)
    plsc.subcore_barrier()

    def wb_drain(b):
        pltpu.make_async_copy(
            bufs[b], out_hbm.at[pl.ds(obase, CHUNK)], sems_w[b]
        ).wait()

    c0 = []
    for b in range(NSLOT):
        t0[b].wait()
        c0.append(
            pltpu.async_copy(comb_sh.at[islice(idx_c, 0, b)], bufs[b],
                             sems_c[b], add=True)
        )
    for b in range(NSLOT):
        c0[b].wait()
        pltpu.async_copy(
            bufs[b], out_hbm.at[pl.ds(obase + b * CHUNK, CHUNK)], sems_w[b]
        )

    def group_body(gg, carry):
        tds = []
        for b in range(NSLOT):
            wb_drain(b)
            tds.append(
                pltpu.async_copy(t_hbm.at[islice(idx_t, gg, b)], bufs[b],
                                 sems_t[b])
            )
        cds = []
        for b in range(NSLOT):
            tds[b].wait()
            cds.append(
                pltpu.async_copy(comb_sh.at[islice(idx_c, gg, b)], bufs[b],
                                 sems_c[b], add=True)
            )
        for b in range(NSLOT):
            cds[b].wait()
            pltpu.async_copy(
                bufs[b],
                out_hbm.at[pl.ds(obase + (gg * NSLOT + b) * CHUNK, CHUNK)],
                sems_w[b],
            )
        return carry

    lax.fori_loop(1, NCHUNK // NSLOT, group_body, 0)
    for b in range(NSLOT):
        wb_drain(b)


@jax.jit
def _sc_run(t_table, s_table, pe, pos, tok, seg):
    mesh = plsc.VectorSubcoreMesh(core_axis_name="c", subcore_axis_name="s")
    f = pl.kernel(
        _sc_body,
        out_type=jax.ShapeDtypeStruct((N, DIM), jnp.float32),
        mesh=mesh,
        scratch_types=[
            pltpu.VMEM_SHARED((NCOMB, DIM), jnp.float32),
            pltpu.VMEM((ROWS_PER_W,), jnp.int32),
            pltpu.VMEM((ROWS_PER_W,), jnp.int32),
            pltpu.VMEM((ROWS_PER_W,), jnp.int32),
            pltpu.VMEM((3, DIM), jnp.float32),
            pltpu.VMEM((CROWS, DIM), jnp.float32),
            [pltpu.VMEM((CHUNK, DIM), jnp.float32) for _ in range(NSLOT)],
            [pltpu.SemaphoreType.DMA for _ in range(NSLOT)],
            [pltpu.SemaphoreType.DMA for _ in range(NSLOT)],
            [pltpu.SemaphoreType.DMA for _ in range(NSLOT)],
        ],
    )
    return f(t_table, s_table, pe, pos, tok, seg)


def kernel(input_batch, segment, t_table, s_table):
    out = _sc_run(
        t_table,
        s_table,
        _PE_NP,
        _POS_NP,
        input_batch.reshape(-1),
        segment.reshape(-1),
    )
    return out.reshape(BATCH, SEQ, DIM)


# speedup vs baseline: 1.0713x; 1.0713x over previous
import functools
import math

import numpy as np
import jax
import jax.numpy as jnp
from jax import lax
from jax.experimental import pallas as pl
from jax.experimental.pallas import tpu as pltpu
from jax.experimental.pallas import tpu_sc as plsc

VOCAB = 100000
DIM = 128
MAX_LEN = 200
BATCH = 1024
SEQ = 200

N = BATCH * SEQ
NW = 32
ROWS_PER_W = N // NW
CHUNK = 128
NCHUNK = ROWS_PER_W // CHUNK
NSLOT = 5
NCOMB = 3 * SEQ
CROWS = 40


def _pe_table_np():
    position = np.arange(MAX_LEN, dtype=np.float32)[:, None]
    div_term = np.exp(
        np.arange(0, DIM, 2, dtype=np.float32) * -(math.log(10000.0) / DIM)
    )
    pe = np.zeros((MAX_LEN, DIM), dtype=np.float32)
    pe[:, 0::2] = np.sin(position * div_term)
    pe[:, 1::2] = np.cos(position * div_term)
    return pe


_PE_NP = _pe_table_np()
_POS_NP = (np.arange(ROWS_PER_W, dtype=np.int32) % SEQ)


def _sc_body(t_hbm, s_hbm, pe_hbm, pos_hbm, tok_hbm, seg_hbm, out_hbm,
             comb_sh, idx_t, idx_c, pos_v, s_v, cbuf, bufs,
             sems_c, sems_t, sems_w):
    cid = lax.axis_index("c")
    sid = lax.axis_index("s")
    wid = sid * 2 + cid
    obase = wid * ROWS_PER_W

    dt = pltpu.async_copy(tok_hbm.at[pl.ds(obase, ROWS_PER_W)], idx_t,
                          sems_t[0])
    dc = pltpu.async_copy(seg_hbm.at[pl.ds(obase, ROWS_PER_W)], idx_c,
                          sems_t[1])
    dp = pltpu.async_copy(pos_hbm, pos_v, sems_t[2])

    @pl.when(sid < 15)
    def _():
        r0 = pl.multiple_of(sid * CROWS, CROWS)
        seg_id = r0 // SEQ
        pos0 = pl.multiple_of(r0 - seg_id * SEQ, CROWS)
        pltpu.sync_copy(pe_hbm.at[pl.ds(pos0, CROWS)], cbuf)
        pltpu.sync_copy(s_hbm, s_v)

        def row_fn(r, c):
            for k in range(DIM // 16):
                sl = pl.ds(k * 16, 16)
                cbuf[r, sl] = cbuf[r, sl] + s_v[seg_id, sl]
            return c

        lax.fori_loop(0, CROWS, row_fn, 0)
        pltpu.sync_copy(cbuf, comb_sh.at[pl.ds(r0, CROWS)])

    dt.wait()
    dc.wait()
    dp.wait()

    def cidx_fn(i, c):
        sl = pl.ds(pl.multiple_of(i * 16, 16), 16)
        idx_c[sl] = idx_c[sl] * SEQ + pos_v[sl]
        return c

    lax.fori_loop(0, ROWS_PER_W // 16, cidx_fn, 0)
    plsc.subcore_barrier()

    def wb_drain(b):
        pltpu.make_async_copy(
            bufs[b], out_hbm.at[pl.ds(obase, CHUNK)], sems_w[b]
        ).wait()

    def group_body(gg, carry):
        g0 = gg * NSLOT

        def islice(ref, b):
            off = pl.multiple_of((g0 + b) * CHUNK, CHUNK)
            return ref.at[pl.ds(off, CHUNK)]

        cds = []
        for b in range(NSLOT):
            @pl.when(gg > 0)
            def _(b=b):
                wb_drain(b)
            cds.append(
                pltpu.async_copy(comb_sh.at[islice(idx_c, b)], bufs[b],
                                 sems_c[b])
            )
        tds = []
        for b in range(NSLOT):
            cds[b].wait()
            tds.append(
                pltpu.async_copy(t_hbm.at[islice(idx_t, b)], bufs[b],
                                 sems_t[b], add=True)
            )
        for b in range(NSLOT):
            tds[b].wait()
            pltpu.async_copy(
                bufs[b],
                out_hbm.at[pl.ds(obase + (g0 + b) * CHUNK, CHUNK)],
                sems_w[b],
            )
        return carry

    lax.fori_loop(0, NCHUNK // NSLOT, group_body, 0)
    for b in range(NSLOT):
        wb_drain(b)


@jax.jit
def _sc_run(t_table, s_table, pe, pos, tok, seg):
    mesh = plsc.VectorSubcoreMesh(core_axis_name="c", subcore_axis_name="s")
    f = pl.kernel(
        _sc_body,
        out_type=jax.ShapeDtypeStruct((N, DIM), jnp.float32),
        mesh=mesh,
        scratch_types=[
            pltpu.VMEM_SHARED((NCOMB, DIM), jnp.float32),
            pltpu.VMEM((ROWS_PER_W,), jnp.int32),
            pltpu.VMEM((ROWS_PER_W,), jnp.int32),
            pltpu.VMEM((ROWS_PER_W,), jnp.int32),
            pltpu.VMEM((3, DIM), jnp.float32),
            pltpu.VMEM((CROWS, DIM), jnp.float32),
            [pltpu.VMEM((CHUNK, DIM), jnp.float32) for _ in range(NSLOT)],
            [pltpu.SemaphoreType.DMA for _ in range(NSLOT)],
            [pltpu.SemaphoreType.DMA for _ in range(NSLOT)],
            [pltpu.SemaphoreType.DMA for _ in range(NSLOT)],
        ],
    )
    return f(t_table, s_table, pe, pos, tok, seg)


def kernel(input_batch, segment, t_table, s_table):
    out = _sc_run(
        t_table,
        s_table,
        _PE_NP,
        _POS_NP,
        input_batch.reshape(-1),
        segment.reshape(-1),
    )
    return out.reshape(BATCH, SEQ, DIM)
